# baseline (speedup 1.0000x reference)
import functools
import math

import jax
import jax.numpy as jnp
from jax import lax
from jax.experimental import pallas as pl
from jax.experimental.pallas import tpu as pltpu
from jax.experimental.pallas import tpu_sc as plsc

VOCAB = 1000000
EMB = 64
PADE = 128
SCALE = math.sqrt(EMB)

NC = 2
NS = 16
NW = NC * NS

BATCH = 4096
TSTEP = 200
BB = 128
NBB = BATCH // BB
NBLK = TSTEP * NBB
BPW = NBLK // NW
LANES = 16
VPR = EMB // LANES
TP = BB + 1
RING = 4


def _emb_kernel_body(table_hbm, tok_hbm, out_hbm, idx_v, gbuf, tbuf,
                     isem, gsem, osem):
    c = lax.axis_index("c")
    s = lax.axis_index("s")
    wid = s * NC + c
    blk0 = wid * BPW

    def blk_t(j):
        beta = blk0 + j
        return beta // NBB, (beta % NBB) * BB

    def fire_idx(j, slot):
        t, b0 = blk_t(j)
        pltpu.async_copy(tok_hbm.at[t, pl.ds(b0, BB)], idx_v.at[slot],
                         isem.at[slot])

    def wait_idx(j, slot):
        t, b0 = blk_t(j)
        pltpu.make_async_copy(tok_hbm.at[t, pl.ds(b0, BB)], idx_v.at[slot],
                              isem.at[slot]).wait()

    def fire_gather(j, slot):
        pltpu.async_copy(table_hbm.at[idx_v.at[slot]], gbuf.at[slot],
                         gsem.at[slot])

    def wait_gather(j, slot):
        pltpu.make_async_copy(table_hbm.at[idx_v.at[slot]], gbuf.at[slot],
                              gsem.at[slot]).wait()

    def fire_store(j, slot):
        t, b0 = blk_t(j)
        pltpu.async_copy(tbuf.at[slot, :, pl.ds(0, BB)],
                         out_hbm.at[t, :, pl.ds(b0, BB)], osem.at[slot])

    def wait_store(j, slot):
        t, b0 = blk_t(j)
        pltpu.make_async_copy(tbuf.at[slot, :, pl.ds(0, BB)],
                              out_hbm.at[t, :, pl.ds(b0, BB)],
                              osem.at[slot]).wait()

    for b in range(RING):
        fire_idx(b, b)
    for b in range(RING - 1):
        wait_idx(b, b)
        fire_gather(b, b)

    lanes = lax.iota(jnp.int32, LANES)

    def step(j, carry):
        slot = j % RING

        @pl.when(j >= RING)
        def _():
            wait_store(j - RING, slot)

        wait_gather(j, slot)

        def tok_body(k, kc):
            for cc in range(VPR):
                v = gbuf[slot, k, pl.ds(cc * LANES, LANES)] * SCALE
                plsc.store_scatter(tbuf.at[slot], [cc * LANES + lanes,
                                                   jnp.full((LANES,), k,
                                                            jnp.int32)], v)
            return kc

        lax.fori_loop(0, BB, tok_body, 0)

        fire_store(j, slot)

        @pl.when(j + RING < BPW)
        def _():
            fire_idx(j + RING, slot)

        @pl.when(j + RING - 1 < BPW)
        def _():
            wait_idx(j + RING - 1, (j + RING - 1) % RING)
            fire_gather(j + RING - 1, (j + RING - 1) % RING)

        return carry

    lax.fori_loop(0, BPW, step, 0)

    for b in range(RING):
        wait_store(BPW - RING + b, (BPW - RING + b) % RING)


@jax.jit
def _emb_lookup(table_padded, tok_t):
    mesh = plsc.VectorSubcoreMesh(core_axis_name="c", subcore_axis_name="s")
    fn = pl.kernel(
        _emb_kernel_body,
        mesh=mesh,
        out_type=jax.ShapeDtypeStruct((TSTEP, EMB, BATCH), jnp.float32),
        scratch_types=[
            pltpu.VMEM((RING, BB), jnp.int32),
            pltpu.VMEM((RING, BB, PADE), jnp.float32),
            pltpu.VMEM((RING, EMB, TP), jnp.float32),
            pltpu.SemaphoreType.DMA((RING,)),
            pltpu.SemaphoreType.DMA((RING,)),
            pltpu.SemaphoreType.DMA((RING,)),
        ],
        compiler_params=pltpu.CompilerParams(use_tc_tiling_on_sc=False,
                                             needs_layout_passes=False),
    )
    return fn(table_padded, tok_t)


def kernel(tokens, table):
    padded = jnp.pad(table, ((0, 0), (0, PADE - EMB)))
    tok_t = jnp.transpose(tokens).astype(jnp.int32)
    out_phys = _emb_lookup(padded, tok_t)
    return jnp.transpose(out_phys, (2, 0, 1))
